# baseline (device time: 596639 ns/iter reference)
import jax
import jax.numpy as jnp
from jax import lax
from jax.experimental import pallas as pl
from jax.experimental.pallas import tpu as pltpu

N_DEV = 16
R_HOPS = 8
L_HOPS = 7
B, SQ, SKV, DM = 2, 256, 256, 512
HQ, DH = 64, 64
HP = 256
HPH = HP // DH
WINDOW = 128

_NT = (((1,), (1,)), ((), ()))


def _body(x_ref, wq_ref, wo_ref, k_hbm, v_hbm, out_ref, wq_full, wo_full,
          k_vmem, v_vmem, kv_sems,
          qs_r, qr_r, os_r, or_r, qs_l, qr_l, os_l, or_l):
    my = lax.axis_index("i")
    left = lax.rem(my + N_DEV - 1, N_DEV)
    right = lax.rem(my + 1, N_DEV)

    k_dma = pltpu.make_async_copy(
        k_hbm.at[pl.ds(my * B, B)], k_vmem, kv_sems.at[0])
    v_dma = pltpu.make_async_copy(
        v_hbm.at[pl.ds(my * B, B)], v_vmem, kv_sems.at[1])
    k_dma.start()
    v_dma.start()

    barrier = pltpu.get_barrier_semaphore()
    pl.semaphore_signal(barrier, inc=1, device_id=(left,),
                        device_id_type=pl.DeviceIdType.MESH)
    pl.semaphore_signal(barrier, inc=1, device_id=(right,),
                        device_id_type=pl.DeviceIdType.MESH)
    pl.semaphore_wait(barrier, 2)

    def rdma(c, tgt, send_sem, recv_sem, is_wq):
        full = wq_full if is_wq else wo_full
        sl = (full.at[:, pl.ds(c * HP, HP)] if is_wq
              else full.at[pl.ds(c * HP, HP), :])
        return pltpu.make_async_remote_copy(
            src_ref=sl, dst_ref=sl,
            send_sem=send_sem, recv_sem=recv_sem,
            device_id=(tgt,), device_id_type=pl.DeviceIdType.MESH,
        )

    qi = lax.broadcasted_iota(jnp.int32, (SQ, SKV), 0)
    ki = lax.broadcasted_iota(jnp.int32, (SQ, SKV), 1)
    band = jnp.abs(qi - ki) <= WINDOW

    xflat = x_ref[...].reshape(B * SQ, DM) * 0.125

    def compute_chunk(j):
        wq_j = wq_full[:, pl.ds(j * HP, HP)]
        wo_j = wo_full[pl.ds(j * HP, HP), :]
        q_all = jnp.dot(xflat, wq_j,
                        preferred_element_type=jnp.float32)
        ctx_rows = []
        for b in range(B):
            qb = q_all[b * SQ:(b + 1) * SQ, :]
            ctx_h = []
            for hh in range(HPH):
                jh = j * HPH + hh
                q = qb[:, hh * DH:(hh + 1) * DH]
                k = k_vmem[b, :, pl.ds(jh, 1), :].reshape(SKV, DH)
                s = lax.dot_general(q, k, _NT,
                                    preferred_element_type=jnp.float32)
                e = jnp.where(band, jnp.exp(s), 0.0)
                w = e / jnp.sum(e, axis=1, keepdims=True)
                v = v_vmem[b, :, pl.ds(jh, 1), :].reshape(SKV, DH)
                ctx_h.append(jnp.dot(w, v,
                                     preferred_element_type=jnp.float32))
            ctx_rows.append(jnp.concatenate(ctx_h, axis=1))
        ctx = jnp.concatenate(ctx_rows, axis=0)
        contrib = jnp.dot(ctx, wo_j,
                          preferred_element_type=jnp.float32)
        out_ref[...] = out_ref[...] + contrib.reshape(B, SQ, DM)

    wq_full[:, pl.ds(my * HP, HP)] = wq_ref[...]
    wo_full[pl.ds(my * HP, HP), :] = wo_ref[...]
    out_ref[...] = jnp.zeros((B, SQ, DM), jnp.float32)

    started = []

    def start_sends(h):
        if h < R_HOPS:
            c = lax.rem(my - h + N_DEV, N_DEV)
            for is_wq, ss, rs in ((True, qs_r, qr_r), (False, os_r, or_r)):
                r = rdma(c, right, ss.at[h], rs.at[h], is_wq)
                r.start()
                started.append(r)
        if h < L_HOPS:
            c = lax.rem(my + h, N_DEV)
            for is_wq, ss, rs in ((True, qs_l, qr_l), (False, os_l, or_l)):
                r = rdma(c, left, ss.at[h], rs.at[h], is_wq)
                r.start()
                started.append(r)

    start_sends(0)
    k_dma.wait()
    v_dma.wait()
    compute_chunk(my)

    for h in range(R_HOPS):
        c_r = lax.rem(my - h - 1 + N_DEV, N_DEV)
        rdma(c_r, left, qs_r.at[h], qr_r.at[h], True).wait_recv()
        rdma(c_r, left, os_r.at[h], or_r.at[h], False).wait_recv()
        if h < L_HOPS:
            c_l = lax.rem(my + h + 1, N_DEV)
            rdma(c_l, right, qs_l.at[h], qr_l.at[h], True).wait_recv()
            rdma(c_l, right, os_l.at[h], or_l.at[h], False).wait_recv()
        start_sends(h + 1)
        compute_chunk(c_r)
        if h < L_HOPS:
            compute_chunk(c_l)

    for r in started:
        r.wait_send()


def kernel(x, Wq, K_ext, V_ext, Wo):
    vmem = pl.BlockSpec(memory_space=pltpu.VMEM)
    hbm = pl.BlockSpec(memory_space=pltpu.HBM)
    return pl.pallas_call(
        _body,
        out_shape=jax.ShapeDtypeStruct((B, SQ, DM), jnp.float32),
        in_specs=[vmem, vmem, vmem, hbm, hbm],
        out_specs=vmem,
        scratch_shapes=[
            pltpu.VMEM((DM, N_DEV * HP), jnp.float32),
            pltpu.VMEM((N_DEV * HP, DM), jnp.float32),
            pltpu.VMEM((B, SKV, HQ, DH), jnp.float32),
            pltpu.VMEM((B, SKV, HQ, DH), jnp.float32),
            pltpu.SemaphoreType.DMA((2,)),
            pltpu.SemaphoreType.DMA((R_HOPS,)),
            pltpu.SemaphoreType.DMA((R_HOPS,)),
            pltpu.SemaphoreType.DMA((R_HOPS,)),
            pltpu.SemaphoreType.DMA((R_HOPS,)),
            pltpu.SemaphoreType.DMA((L_HOPS,)),
            pltpu.SemaphoreType.DMA((L_HOPS,)),
            pltpu.SemaphoreType.DMA((L_HOPS,)),
            pltpu.SemaphoreType.DMA((L_HOPS,)),
        ],
        compiler_params=pltpu.CompilerParams(
            collective_id=0,
            vmem_limit_bytes=100 * 1024 * 1024,
        ),
    )(x, Wq, Wo, K_ext, V_ext)


# device time: 142527 ns/iter; 4.1861x vs baseline; 4.1861x over previous
import jax
import jax.numpy as jnp
from jax import lax
from jax.experimental import pallas as pl
from jax.experimental.pallas import tpu as pltpu

N_DEV = 16
R_HOPS = 8
L_HOPS = 7
B, SQ, SKV, DM = 2, 256, 256, 512
HQ, DH = 64, 64
HP = 256
HPH = HP // DH
WINDOW = 128


def _body(x_ref, wq_ref, wo_ref, kt_ref, vt_ref, out_ref, wq_full, wo_full,
          qs_r, qr_r, os_r, or_r, qs_l, qr_l, os_l, or_l):
    my = lax.axis_index("i")
    left = lax.rem(my + N_DEV - 1, N_DEV)
    right = lax.rem(my + 1, N_DEV)

    barrier = pltpu.get_barrier_semaphore()
    pl.semaphore_signal(barrier, inc=1, device_id=(left,),
                        device_id_type=pl.DeviceIdType.MESH)
    pl.semaphore_signal(barrier, inc=1, device_id=(right,),
                        device_id_type=pl.DeviceIdType.MESH)
    pl.semaphore_wait(barrier, 2)

    def rdma(c, tgt, send_sem, recv_sem, is_wq):
        full = wq_full if is_wq else wo_full
        sl = (full.at[:, pl.ds(c * HP, HP)] if is_wq
              else full.at[pl.ds(c * HP, HP), :])
        return pltpu.make_async_remote_copy(
            src_ref=sl, dst_ref=sl,
            send_sem=send_sem, recv_sem=recv_sem,
            device_id=(tgt,), device_id_type=pl.DeviceIdType.MESH,
        )

    qi = lax.broadcasted_iota(jnp.int32, (SQ, SKV), 0)
    ki = lax.broadcasted_iota(jnp.int32, (SQ, SKV), 1)
    band = jnp.abs(qi - ki) <= WINDOW

    xflat = x_ref[...].reshape(B * SQ, DM) * 0.125

    def compute_chunk(j):
        wq_j = wq_full[:, pl.ds(j * HP, HP)]
        wo_j = wo_full[pl.ds(j * HP, HP), :]
        q_all = jnp.dot(xflat, wq_j,
                        preferred_element_type=jnp.float32)
        ctx_rows = []
        for b in range(B):
            qb = q_all[b * SQ:(b + 1) * SQ, :]
            ctx_h = []
            for hh in range(HPH):
                jh = j * HPH + hh
                q = qb[:, hh * DH:(hh + 1) * DH]
                kt = kt_ref[b, pl.ds(jh, 1), :, :].reshape(DH, SKV)
                s = jnp.dot(q, kt,
                            preferred_element_type=jnp.float32)
                e = jnp.where(band, jnp.exp(s), 0.0)
                w = e / jnp.sum(e, axis=1, keepdims=True)
                v = vt_ref[b, pl.ds(jh, 1), :, :].reshape(SKV, DH)
                ctx_h.append(jnp.dot(w, v,
                                     preferred_element_type=jnp.float32))
            ctx_rows.append(jnp.concatenate(ctx_h, axis=1))
        ctx = jnp.concatenate(ctx_rows, axis=0)
        contrib = jnp.dot(ctx, wo_j,
                          preferred_element_type=jnp.float32)
        out_ref[...] = out_ref[...] + contrib.reshape(B, SQ, DM)

    wq_full[:, pl.ds(my * HP, HP)] = wq_ref[...]
    wo_full[pl.ds(my * HP, HP), :] = wo_ref[...]
    out_ref[...] = jnp.zeros((B, SQ, DM), jnp.float32)

    started = []

    def start_right(h):
        c = lax.rem(my - h + N_DEV, N_DEV)
        for is_wq, ss, rs in ((True, qs_r, qr_r), (False, os_r, or_r)):
            r = rdma(c, right, ss.at[h], rs.at[h], is_wq)
            r.start()
            started.append(r)

    def start_left(h):
        c = lax.rem(my + h, N_DEV)
        for is_wq, ss, rs in ((True, qs_l, qr_l), (False, os_l, or_l)):
            r = rdma(c, left, ss.at[h], rs.at[h], is_wq)
            r.start()
            started.append(r)

    start_right(0)
    start_left(0)
    compute_chunk(my)

    for h in range(R_HOPS):
        c_r = lax.rem(my - h - 1 + N_DEV, N_DEV)
        rdma(c_r, left, qs_r.at[h], qr_r.at[h], True).wait_recv()
        rdma(c_r, left, os_r.at[h], or_r.at[h], False).wait_recv()
        if h + 1 < R_HOPS:
            start_right(h + 1)
        if h < L_HOPS:
            c_l = lax.rem(my + h + 1, N_DEV)
            rdma(c_l, right, qs_l.at[h], qr_l.at[h], True).wait_recv()
            rdma(c_l, right, os_l.at[h], or_l.at[h], False).wait_recv()
            if h + 1 < L_HOPS:
                start_left(h + 1)
        compute_chunk(c_r)
        if h < L_HOPS:
            compute_chunk(c_l)

    for r in started:
        r.wait_send()


def kernel(x, Wq, K_ext, V_ext, Wo):
    my = lax.axis_index("i")
    K_loc = lax.dynamic_slice_in_dim(K_ext, my * B, B, axis=0)
    V_loc = lax.dynamic_slice_in_dim(V_ext, my * B, B, axis=0)
    KT = K_loc.transpose(0, 2, 3, 1)
    VT = V_loc.transpose(0, 2, 1, 3)

    vmem = pl.BlockSpec(memory_space=pltpu.VMEM)
    return pl.pallas_call(
        _body,
        out_shape=jax.ShapeDtypeStruct((B, SQ, DM), jnp.float32),
        in_specs=[vmem] * 5,
        out_specs=vmem,
        scratch_shapes=[
            pltpu.VMEM((DM, N_DEV * HP), jnp.float32),
            pltpu.VMEM((N_DEV * HP, DM), jnp.float32),
            pltpu.SemaphoreType.DMA((R_HOPS,)),
            pltpu.SemaphoreType.DMA((R_HOPS,)),
            pltpu.SemaphoreType.DMA((R_HOPS,)),
            pltpu.SemaphoreType.DMA((R_HOPS,)),
            pltpu.SemaphoreType.DMA((L_HOPS,)),
            pltpu.SemaphoreType.DMA((L_HOPS,)),
            pltpu.SemaphoreType.DMA((L_HOPS,)),
            pltpu.SemaphoreType.DMA((L_HOPS,)),
        ],
        compiler_params=pltpu.CompilerParams(
            collective_id=0,
            vmem_limit_bytes=100 * 1024 * 1024,
        ),
    )(x, Wq, Wo, KT, VT)


# device time: 140752 ns/iter; 4.2389x vs baseline; 1.0126x over previous
import jax
import jax.numpy as jnp
from jax import lax
from jax.experimental import pallas as pl
from jax.experimental.pallas import tpu as pltpu

N_DEV = 16
R_HOPS = 8
L_HOPS = 7
B, SQ, SKV, DM = 2, 256, 256, 512
HQ, DH = 64, 64
HP = 256
HPH = HP // DH
WINDOW = 128

CYCLE = [0, 1, 2, 3, 7, 6, 5, 9, 10, 11, 15, 14, 13, 12, 8, 4]
INV = [0] * N_DEV
for _p, _l in enumerate(CYCLE):
    INV[_l] = _p
LEFT_T = [CYCLE[(INV[l] - 1) % N_DEV] for l in range(N_DEV)]
RIGHT_T = [CYCLE[(INV[l] + 1) % N_DEV] for l in range(N_DEV)]
SEND_R = [[CYCLE[(INV[l] - h) % N_DEV] for l in range(N_DEV)]
          for h in range(R_HOPS)]
RECV_R = [[CYCLE[(INV[l] - h - 1) % N_DEV] for l in range(N_DEV)]
          for h in range(R_HOPS)]
SEND_L = [[CYCLE[(INV[l] + h) % N_DEV] for l in range(N_DEV)]
          for h in range(L_HOPS)]
RECV_L = [[CYCLE[(INV[l] + h + 1) % N_DEV] for l in range(N_DEV)]
          for h in range(L_HOPS)]


def _body(x_ref, wq_ref, wo_ref, kt_ref, vt_ref, out_ref, wq_full, wo_full,
          qs_r, qr_r, os_r, or_r, qs_l, qr_l, os_l, or_l):
    my = lax.axis_index("i")

    def sel(table):
        r = jnp.int32(table[0])
        for p in range(1, N_DEV):
            r = jnp.where(my == p, jnp.int32(table[p]), r)
        return r

    left = sel(LEFT_T)
    right = sel(RIGHT_T)

    barrier = pltpu.get_barrier_semaphore()
    pl.semaphore_signal(barrier, inc=1, device_id=(left,),
                        device_id_type=pl.DeviceIdType.MESH)
    pl.semaphore_signal(barrier, inc=1, device_id=(right,),
                        device_id_type=pl.DeviceIdType.MESH)
    pl.semaphore_wait(barrier, 2)

    def rdma(c, tgt, send_sem, recv_sem, is_wq):
        full = wq_full if is_wq else wo_full
        sl = (full.at[:, pl.ds(c * HP, HP)] if is_wq
              else full.at[pl.ds(c * HP, HP), :])
        return pltpu.make_async_remote_copy(
            src_ref=sl, dst_ref=sl,
            send_sem=send_sem, recv_sem=recv_sem,
            device_id=(tgt,), device_id_type=pl.DeviceIdType.MESH,
        )

    qi = lax.broadcasted_iota(jnp.int32, (SQ, SKV), 0)
    ki = lax.broadcasted_iota(jnp.int32, (SQ, SKV), 1)
    band = jnp.abs(qi - ki) <= WINDOW

    xflat = x_ref[...].reshape(B * SQ, DM) * 0.125

    def compute_chunk(j):
        wq_j = wq_full[:, pl.ds(j * HP, HP)]
        wo_j = wo_full[pl.ds(j * HP, HP), :]
        q_all = jnp.dot(xflat, wq_j,
                        preferred_element_type=jnp.float32)
        ctx_rows = []
        for b in range(B):
            qb = q_all[b * SQ:(b + 1) * SQ, :]
            ctx_h = []
            for hh in range(HPH):
                jh = j * HPH + hh
                q = qb[:, hh * DH:(hh + 1) * DH]
                kt = kt_ref[b, pl.ds(jh, 1), :, :].reshape(DH, SKV)
                s = jnp.dot(q, kt,
                            preferred_element_type=jnp.float32)
                e = jnp.where(band, jnp.exp(s), 0.0)
                w = e / jnp.sum(e, axis=1, keepdims=True)
                v = vt_ref[b, pl.ds(jh, 1), :, :].reshape(SKV, DH)
                ctx_h.append(jnp.dot(w, v,
                                     preferred_element_type=jnp.float32))
            ctx_rows.append(jnp.concatenate(ctx_h, axis=1))
        ctx = jnp.concatenate(ctx_rows, axis=0)
        contrib = jnp.dot(ctx, wo_j,
                          preferred_element_type=jnp.float32)
        out_ref[...] = out_ref[...] + contrib.reshape(B, SQ, DM)

    wq_full[:, pl.ds(my * HP, HP)] = wq_ref[...]
    wo_full[pl.ds(my * HP, HP), :] = wo_ref[...]
    out_ref[...] = jnp.zeros((B, SQ, DM), jnp.float32)

    started = []

    def start_right(h):
        c = sel(SEND_R[h])
        for is_wq, ss, rs in ((True, qs_r, qr_r), (False, os_r, or_r)):
            r = rdma(c, right, ss.at[h], rs.at[h], is_wq)
            r.start()
            started.append(r)

    def start_left(h):
        c = sel(SEND_L[h])
        for is_wq, ss, rs in ((True, qs_l, qr_l), (False, os_l, or_l)):
            r = rdma(c, left, ss.at[h], rs.at[h], is_wq)
            r.start()
            started.append(r)

    start_right(0)
    start_left(0)
    compute_chunk(my)

    for h in range(R_HOPS):
        c_r = sel(RECV_R[h])
        rdma(c_r, left, qs_r.at[h], qr_r.at[h], True).wait_recv()
        rdma(c_r, left, os_r.at[h], or_r.at[h], False).wait_recv()
        if h + 1 < R_HOPS:
            start_right(h + 1)
        if h < L_HOPS:
            c_l = sel(RECV_L[h])
            rdma(c_l, right, qs_l.at[h], qr_l.at[h], True).wait_recv()
            rdma(c_l, right, os_l.at[h], or_l.at[h], False).wait_recv()
            if h + 1 < L_HOPS:
                start_left(h + 1)
        compute_chunk(c_r)
        if h < L_HOPS:
            compute_chunk(c_l)

    for r in started:
        r.wait_send()


def kernel(x, Wq, K_ext, V_ext, Wo):
    my = lax.axis_index("i")
    K_loc = lax.dynamic_slice_in_dim(K_ext, my * B, B, axis=0)
    V_loc = lax.dynamic_slice_in_dim(V_ext, my * B, B, axis=0)
    KT = K_loc.transpose(0, 2, 3, 1)
    VT = V_loc.transpose(0, 2, 1, 3)

    vmem = pl.BlockSpec(memory_space=pltpu.VMEM)
    return pl.pallas_call(
        _body,
        out_shape=jax.ShapeDtypeStruct((B, SQ, DM), jnp.float32),
        in_specs=[vmem] * 5,
        out_specs=vmem,
        scratch_shapes=[
            pltpu.VMEM((DM, N_DEV * HP), jnp.float32),
            pltpu.VMEM((N_DEV * HP, DM), jnp.float32),
            pltpu.SemaphoreType.DMA((R_HOPS,)),
            pltpu.SemaphoreType.DMA((R_HOPS,)),
            pltpu.SemaphoreType.DMA((R_HOPS,)),
            pltpu.SemaphoreType.DMA((R_HOPS,)),
            pltpu.SemaphoreType.DMA((L_HOPS,)),
            pltpu.SemaphoreType.DMA((L_HOPS,)),
            pltpu.SemaphoreType.DMA((L_HOPS,)),
            pltpu.SemaphoreType.DMA((L_HOPS,)),
        ],
        compiler_params=pltpu.CompilerParams(
            collective_id=0,
            vmem_limit_bytes=100 * 1024 * 1024,
        ),
    )(x, Wq, Wo, KT, VT)


# device time: 89700 ns/iter; 6.6515x vs baseline; 1.5691x over previous
import jax
import jax.numpy as jnp
from jax import lax
from jax.experimental import pallas as pl
from jax.experimental.pallas import tpu as pltpu

N_DEV = 16
R_HOPS = 8
L_HOPS = 7
B, SQ, SKV, DM = 2, 256, 256, 512
HQ, DH = 64, 64
HP = 256
HPH = HP // DH
WINDOW = 128

CYCLE = [0, 1, 2, 3, 7, 6, 5, 9, 10, 11, 15, 14, 13, 12, 8, 4]
INV = [0] * N_DEV
for _p, _l in enumerate(CYCLE):
    INV[_l] = _p
LEFT_T = [CYCLE[(INV[l] - 1) % N_DEV] for l in range(N_DEV)]
RIGHT_T = [CYCLE[(INV[l] + 1) % N_DEV] for l in range(N_DEV)]
SEND_R = [[CYCLE[(INV[l] - h) % N_DEV] for l in range(N_DEV)]
          for h in range(R_HOPS)]
RECV_R = [[CYCLE[(INV[l] - h - 1) % N_DEV] for l in range(N_DEV)]
          for h in range(R_HOPS)]
SEND_L = [[CYCLE[(INV[l] + h) % N_DEV] for l in range(N_DEV)]
          for h in range(L_HOPS)]
RECV_L = [[CYCLE[(INV[l] + h + 1) % N_DEV] for l in range(N_DEV)]
          for h in range(L_HOPS)]


def _body(x_ref, wq_ref, wo_ref, kt_ref, vt_ref, out_ref, wq_full, wo_full,
          qs_r, qr_r, os_r, or_r, qs_l, qr_l, os_l, or_l):
    my = lax.axis_index("i")

    def sel(table):
        r = jnp.int32(table[0])
        for p in range(1, N_DEV):
            r = jnp.where(my == p, jnp.int32(table[p]), r)
        return r

    left = sel(LEFT_T)
    right = sel(RIGHT_T)

    barrier = pltpu.get_barrier_semaphore()
    pl.semaphore_signal(barrier, inc=1, device_id=(left,),
                        device_id_type=pl.DeviceIdType.MESH)
    pl.semaphore_signal(barrier, inc=1, device_id=(right,),
                        device_id_type=pl.DeviceIdType.MESH)
    pl.semaphore_wait(barrier, 2)

    def rdma(c, tgt, send_sem, recv_sem, is_wq):
        full = wq_full if is_wq else wo_full
        sl = (full.at[:, pl.ds(c * HP, HP)] if is_wq
              else full.at[pl.ds(c * HP, HP), :])
        return pltpu.make_async_remote_copy(
            src_ref=sl, dst_ref=sl,
            send_sem=send_sem, recv_sem=recv_sem,
            device_id=(tgt,), device_id_type=pl.DeviceIdType.MESH,
        )

    qi = lax.broadcasted_iota(jnp.int32, (SQ, SKV), 0)
    ki = lax.broadcasted_iota(jnp.int32, (SQ, SKV), 1)
    band = jnp.abs(qi - ki) <= WINDOW

    xflat = x_ref[...].reshape(B * SQ, DM) * jnp.bfloat16(0.125)

    def compute_chunk(j):
        wq_j = wq_full[:, pl.ds(j * HP, HP)]
        wo_j = wo_full[pl.ds(j * HP, HP), :]
        q_all = jnp.dot(xflat, wq_j,
                        preferred_element_type=jnp.float32
                        ).astype(jnp.bfloat16)
        ctx_rows = []
        for b in range(B):
            qb = q_all[b * SQ:(b + 1) * SQ, :]
            ctx_h = []
            for hh in range(HPH):
                jh = j * HPH + hh
                q = qb[:, hh * DH:(hh + 1) * DH]
                kt = kt_ref[b, pl.ds(jh, 1), :, :].reshape(DH, SKV)
                s = jnp.dot(q, kt,
                            preferred_element_type=jnp.float32)
                e = jnp.where(band, jnp.exp(s), 0.0)
                w = (e / jnp.sum(e, axis=1, keepdims=True)).astype(jnp.bfloat16)
                v = vt_ref[b, pl.ds(jh, 1), :, :].reshape(SKV, DH)
                ctx_h.append(jnp.dot(w, v,
                                     preferred_element_type=jnp.float32))
            ctx_rows.append(jnp.concatenate(ctx_h, axis=1))
        ctx = jnp.concatenate(ctx_rows, axis=0).astype(jnp.bfloat16)
        contrib = jnp.dot(ctx, wo_j,
                          preferred_element_type=jnp.float32)
        out_ref[...] = out_ref[...] + contrib.reshape(B, SQ, DM)

    wq_full[:, pl.ds(my * HP, HP)] = wq_ref[...]
    wo_full[pl.ds(my * HP, HP), :] = wo_ref[...]
    out_ref[...] = jnp.zeros((B, SQ, DM), jnp.float32)

    started = []

    def start_right(h):
        c = sel(SEND_R[h])
        for is_wq, ss, rs in ((True, qs_r, qr_r), (False, os_r, or_r)):
            r = rdma(c, right, ss.at[h], rs.at[h], is_wq)
            r.start()
            started.append(r)

    def start_left(h):
        c = sel(SEND_L[h])
        for is_wq, ss, rs in ((True, qs_l, qr_l), (False, os_l, or_l)):
            r = rdma(c, left, ss.at[h], rs.at[h], is_wq)
            r.start()
            started.append(r)

    start_right(0)
    start_left(0)
    compute_chunk(my)

    for h in range(R_HOPS):
        c_r = sel(RECV_R[h])
        rdma(c_r, left, qs_r.at[h], qr_r.at[h], True).wait_recv()
        rdma(c_r, left, os_r.at[h], or_r.at[h], False).wait_recv()
        if h + 1 < R_HOPS:
            start_right(h + 1)
        if h < L_HOPS:
            c_l = sel(RECV_L[h])
            rdma(c_l, right, qs_l.at[h], qr_l.at[h], True).wait_recv()
            rdma(c_l, right, os_l.at[h], or_l.at[h], False).wait_recv()
            if h + 1 < L_HOPS:
                start_left(h + 1)
        compute_chunk(c_r)
        if h < L_HOPS:
            compute_chunk(c_l)

    for r in started:
        r.wait_send()


def kernel(x, Wq, K_ext, V_ext, Wo):
    my = lax.axis_index("i")
    K_loc = lax.dynamic_slice_in_dim(K_ext, my * B, B, axis=0)
    V_loc = lax.dynamic_slice_in_dim(V_ext, my * B, B, axis=0)
    KT = K_loc.transpose(0, 2, 3, 1).astype(jnp.bfloat16)
    VT = V_loc.transpose(0, 2, 1, 3).astype(jnp.bfloat16)
    x = x.astype(jnp.bfloat16)
    Wq = Wq.astype(jnp.bfloat16)
    Wo = Wo.astype(jnp.bfloat16)

    vmem = pl.BlockSpec(memory_space=pltpu.VMEM)
    return pl.pallas_call(
        _body,
        out_shape=jax.ShapeDtypeStruct((B, SQ, DM), jnp.float32),
        in_specs=[vmem] * 5,
        out_specs=vmem,
        scratch_shapes=[
            pltpu.VMEM((DM, N_DEV * HP), jnp.bfloat16),
            pltpu.VMEM((N_DEV * HP, DM), jnp.bfloat16),
            pltpu.SemaphoreType.DMA((R_HOPS,)),
            pltpu.SemaphoreType.DMA((R_HOPS,)),
            pltpu.SemaphoreType.DMA((R_HOPS,)),
            pltpu.SemaphoreType.DMA((R_HOPS,)),
            pltpu.SemaphoreType.DMA((L_HOPS,)),
            pltpu.SemaphoreType.DMA((L_HOPS,)),
            pltpu.SemaphoreType.DMA((L_HOPS,)),
            pltpu.SemaphoreType.DMA((L_HOPS,)),
        ],
        compiler_params=pltpu.CompilerParams(
            collective_id=0,
            vmem_limit_bytes=100 * 1024 * 1024,
        ),
    )(x, Wq, Wo, KT, VT)


# device time: 77361 ns/iter; 7.7124x vs baseline; 1.1595x over previous
import jax
import jax.numpy as jnp
from jax import lax
from jax.experimental import pallas as pl
from jax.experimental.pallas import tpu as pltpu

N_DEV = 16
R_HOPS = 8
L_HOPS = 7
B, SQ, SKV, DM = 2, 256, 256, 512
HQ, DH = 64, 64
HP = 256
HPH = HP // DH
WINDOW = 128

CYCLE = [0, 1, 2, 3, 7, 6, 5, 9, 10, 11, 15, 14, 13, 12, 8, 4]
INV = [0] * N_DEV
for _p, _l in enumerate(CYCLE):
    INV[_l] = _p
LEFT_T = [CYCLE[(INV[l] - 1) % N_DEV] for l in range(N_DEV)]
RIGHT_T = [CYCLE[(INV[l] + 1) % N_DEV] for l in range(N_DEV)]
SEND_R = [[CYCLE[(INV[l] - h) % N_DEV] for l in range(N_DEV)]
          for h in range(R_HOPS)]
RECV_R = [[CYCLE[(INV[l] - h - 1) % N_DEV] for l in range(N_DEV)]
          for h in range(R_HOPS)]
SEND_L = [[CYCLE[(INV[l] + h) % N_DEV] for l in range(N_DEV)]
          for h in range(L_HOPS)]
RECV_L = [[CYCLE[(INV[l] + h + 1) % N_DEV] for l in range(N_DEV)]
          for h in range(L_HOPS)]


def _body(x_ref, wq_ref, wo_ref, kt_ref, vt_ref, out_ref, wq_full, wo_full,
          qs_r, qr_r, os_r, or_r, qs_l, qr_l, os_l, or_l):
    my = lax.axis_index("i")

    def sel(table):
        r = jnp.int32(table[0])
        for p in range(1, N_DEV):
            r = jnp.where(my == p, jnp.int32(table[p]), r)
        return r

    left = sel(LEFT_T)
    right = sel(RIGHT_T)

    barrier = pltpu.get_barrier_semaphore()
    pl.semaphore_signal(barrier, inc=1, device_id=(left,),
                        device_id_type=pl.DeviceIdType.MESH)
    pl.semaphore_signal(barrier, inc=1, device_id=(right,),
                        device_id_type=pl.DeviceIdType.MESH)
    pl.semaphore_wait(barrier, 2)

    def rdma(c, tgt, send_sem, recv_sem, is_wq, sub):
        if is_wq:
            sl = wq_full.at[pl.ds(sub * DM // 2, DM // 2),
                            pl.ds(c * HP, HP)]
        else:
            sl = wo_full.at[pl.ds(c * HP, HP),
                            pl.ds(sub * DM // 2, DM // 2)]
        return pltpu.make_async_remote_copy(
            src_ref=sl, dst_ref=sl,
            send_sem=send_sem, recv_sem=recv_sem,
            device_id=(tgt,), device_id_type=pl.DeviceIdType.MESH,
        )

    qi = lax.broadcasted_iota(jnp.int32, (SQ, SKV), 0)
    ki = lax.broadcasted_iota(jnp.int32, (SQ, SKV), 1)
    band = jnp.abs(qi - ki) <= WINDOW

    xflat = x_ref[...].reshape(B * SQ, DM) * jnp.bfloat16(0.125)

    def compute_chunk(j):
        wq_j = wq_full[:, pl.ds(j * HP, HP)]
        wo_j = wo_full[pl.ds(j * HP, HP), :]
        q_all = jnp.dot(xflat, wq_j,
                        preferred_element_type=jnp.float32
                        ).astype(jnp.bfloat16)
        ctx_rows = []
        for b in range(B):
            qb = q_all[b * SQ:(b + 1) * SQ, :]
            ctx_h = []
            for hh in range(HPH):
                jh = j * HPH + hh
                q = qb[:, hh * DH:(hh + 1) * DH]
                kt = kt_ref[b, pl.ds(jh, 1), :, :].reshape(DH, SKV)
                s = jnp.dot(q, kt,
                            preferred_element_type=jnp.float32)
                e = jnp.where(band, jnp.exp(s), 0.0)
                w = (e / jnp.sum(e, axis=1, keepdims=True)).astype(jnp.bfloat16)
                v = vt_ref[b, pl.ds(jh, 1), :, :].reshape(SKV, DH)
                ctx_h.append(jnp.dot(w, v,
                                     preferred_element_type=jnp.float32))
            ctx_rows.append(jnp.concatenate(ctx_h, axis=1))
        ctx = jnp.concatenate(ctx_rows, axis=0).astype(jnp.bfloat16)
        contrib = jnp.dot(ctx, wo_j,
                          preferred_element_type=jnp.float32)
        out_ref[...] = out_ref[...] + contrib.reshape(B, SQ, DM)

    wq_full[:, pl.ds(my * HP, HP)] = wq_ref[...]
    wo_full[pl.ds(my * HP, HP), :] = wo_ref[...]
    out_ref[...] = jnp.zeros((B, SQ, DM), jnp.float32)

    started = []

    def start_right(h, sub):
        c = sel(SEND_R[h])
        for is_wq, ss, rs in ((True, qs_r, qr_r), (False, os_r, or_r)):
            r = rdma(c, right, ss.at[h, sub], rs.at[h, sub], is_wq, sub)
            r.start()
            started.append(r)

    def start_left(h, sub):
        c = sel(SEND_L[h])
        for is_wq, ss, rs in ((True, qs_l, qr_l), (False, os_l, or_l)):
            r = rdma(c, left, ss.at[h, sub], rs.at[h, sub], is_wq, sub)
            r.start()
            started.append(r)

    for sub in range(2):
        start_right(0, sub)
        start_left(0, sub)
    compute_chunk(my)

    for h in range(R_HOPS):
        c_r = sel(RECV_R[h])
        c_l = sel(RECV_L[h]) if h < L_HOPS else None
        for sub in range(2):
            rdma(c_r, left, qs_r.at[h, sub], qr_r.at[h, sub],
                 True, sub).wait_recv()
            rdma(c_r, left, os_r.at[h, sub], or_r.at[h, sub],
                 False, sub).wait_recv()
            if h + 1 < R_HOPS:
                start_right(h + 1, sub)
            if h < L_HOPS:
                rdma(c_l, right, qs_l.at[h, sub], qr_l.at[h, sub],
                     True, sub).wait_recv()
                rdma(c_l, right, os_l.at[h, sub], or_l.at[h, sub],
                     False, sub).wait_recv()
                if h + 1 < L_HOPS:
                    start_left(h + 1, sub)
        compute_chunk(c_r)
        if h < L_HOPS:
            compute_chunk(c_l)

    for r in started:
        r.wait_send()


def kernel(x, Wq, K_ext, V_ext, Wo):
    my = lax.axis_index("i")
    K_loc = lax.dynamic_slice_in_dim(K_ext, my * B, B, axis=0)
    V_loc = lax.dynamic_slice_in_dim(V_ext, my * B, B, axis=0)
    KT = K_loc.transpose(0, 2, 3, 1).astype(jnp.bfloat16)
    VT = V_loc.transpose(0, 2, 1, 3).astype(jnp.bfloat16)
    x = x.astype(jnp.bfloat16)
    Wq = Wq.astype(jnp.bfloat16)
    Wo = Wo.astype(jnp.bfloat16)

    vmem = pl.BlockSpec(memory_space=pltpu.VMEM)
    return pl.pallas_call(
        _body,
        out_shape=jax.ShapeDtypeStruct((B, SQ, DM), jnp.float32),
        in_specs=[vmem] * 5,
        out_specs=vmem,
        scratch_shapes=[
            pltpu.VMEM((DM, N_DEV * HP), jnp.bfloat16),
            pltpu.VMEM((N_DEV * HP, DM), jnp.bfloat16),
            pltpu.SemaphoreType.DMA((R_HOPS, 2)),
            pltpu.SemaphoreType.DMA((R_HOPS, 2)),
            pltpu.SemaphoreType.DMA((R_HOPS, 2)),
            pltpu.SemaphoreType.DMA((R_HOPS, 2)),
            pltpu.SemaphoreType.DMA((L_HOPS, 2)),
            pltpu.SemaphoreType.DMA((L_HOPS, 2)),
            pltpu.SemaphoreType.DMA((L_HOPS, 2)),
            pltpu.SemaphoreType.DMA((L_HOPS, 2)),
        ],
        compiler_params=pltpu.CompilerParams(
            collective_id=0,
            vmem_limit_bytes=100 * 1024 * 1024,
        ),
    )(x, Wq, Wo, KT, VT)
